# baseline (speedup 1.0000x reference)
import jax
import jax.numpy as jnp
from jax.experimental import pallas as pl

HEIGHT = 128
WIDTH = 128
SIZE = 8
NEURONS = 16
NUM_W = WIDTH - (SIZE - 1)
NUM_H = HEIGHT - (SIZE - 1)
NUM_ZONES = NUM_H * NUM_W
PATCH = SIZE * SIZE
RPB = 11


def _tc_body(x_ref, *refs):
    w_refs = refs[:RPB]
    o_ref = refs[RPB]
    i = pl.program_id(0)
    base = i * RPB
    xs = x_ref[pl.ds(base, RPB + SIZE - 1), :]

    li = jax.lax.broadcasted_iota(jnp.int32, (NEURONS * PATCH, NEURONS), 0)
    ni = jax.lax.broadcasted_iota(jnp.int32, (NEURONS * PATCH, NEURONS), 1)
    S = (li // PATCH == ni).astype(jnp.float32)

    for rr in range(RPB):
        segs = []
        for dr in range(SIZE):
            row = xs[rr + dr:rr + dr + 1, :]
            for dc in range(SIZE):
                segs.append(row[:, dc:dc + NUM_W])
        PT = jnp.concatenate(segs, axis=0)
        P = PT.T
        Pt = jnp.tile(P, (1, NEURONS))
        prod = w_refs[rr][0] * Pt
        resp = jnp.dot(prod, S, preferred_element_type=jnp.float32,
                       precision=jax.lax.Precision.HIGHEST)
        m = jnp.max(resp, axis=1, keepdims=True)
        o_ref[rr] = jnp.where(resp >= m, resp, 0.0)


def kernel(x, W):
    W3 = W.reshape(NUM_H, NUM_W, NEURONS * PATCH)
    w_specs = [
        pl.BlockSpec((1, NUM_W, NEURONS * PATCH),
                     lambda i, k=k: (i * RPB + k, 0, 0))
        for k in range(RPB)
    ]
    out = pl.pallas_call(
        _tc_body,
        grid=(NUM_H // RPB,),
        in_specs=[pl.BlockSpec((HEIGHT, WIDTH), lambda i: (0, 0))] + w_specs,
        out_specs=pl.BlockSpec((RPB, NUM_W, NEURONS), lambda i: (i, 0, 0)),
        out_shape=jax.ShapeDtypeStruct((NUM_H, NUM_W, NEURONS), jnp.float32),
    )(x, *([W3] * RPB))
    return out.reshape(NUM_ZONES, NEURONS)


# speedup vs baseline: 1.4325x; 1.4325x over previous
import jax
import jax.numpy as jnp
from jax.experimental import pallas as pl

HEIGHT = 128
WIDTH = 128
SIZE = 8
NEURONS = 16
NUM_W = WIDTH - (SIZE - 1)
NUM_H = HEIGHT - (SIZE - 1)
NUM_ZONES = NUM_H * NUM_W
PATCH = SIZE * SIZE
RPB = 11


def _tc_body(x_ref, w_ref, o_ref):
    i = pl.program_id(0)
    base = i * RPB
    xs = x_ref[pl.ds(base, RPB + SIZE - 1), :]

    for rr in range(RPB):
        segs = []
        for dr in range(SIZE):
            row = xs[rr + dr:rr + dr + 1, :]
            for dc in range(SIZE):
                segs.append(row[:, dc:dc + NUM_W])
        PT = jnp.concatenate(segs, axis=0)
        P = PT.T
        prod = w_ref[rr] * P[:, None, :]
        resp = jnp.sum(prod, axis=2)
        m = jnp.max(resp, axis=1, keepdims=True)
        o_ref[rr] = jnp.where(resp >= m, resp, 0.0)


def kernel(x, W):
    W4 = W.reshape(NUM_H, NUM_W, NEURONS, PATCH)
    out = pl.pallas_call(
        _tc_body,
        grid=(NUM_H // RPB,),
        in_specs=[
            pl.BlockSpec((HEIGHT, WIDTH), lambda i: (0, 0)),
            pl.BlockSpec((RPB, NUM_W, NEURONS, PATCH), lambda i: (i, 0, 0, 0)),
        ],
        out_specs=pl.BlockSpec((RPB, NUM_W, NEURONS), lambda i: (i, 0, 0)),
        out_shape=jax.ShapeDtypeStruct((NUM_H, NUM_W, NEURONS), jnp.float32),
    )(x, W4)
    return out.reshape(NUM_ZONES, NEURONS)
